# baseline (speedup 1.0000x reference)
import jax
import jax.numpy as jnp
from jax import lax
from jax.experimental import pallas as pl
from jax.experimental.pallas import tpu as pltpu
from jax.experimental.pallas import tpu_sc as plsc

N_NODES = 10000
N_EDGES = 320000
EMB = 128
NUM_COMBO = 15
N_COMBO_ROWS = N_NODES * NUM_COMBO

NC = 2
NS = 16
NW = NC * NS
EDGE_BATCH = 80
N_BATCH = N_EDGES // EDGE_BATCH
NB_PER_TILE = N_BATCH // NW
CHUNK = 25
N_CHUNKS = NB_PER_TILE // CHUNK
NSLOT = 4
N_PAD = 10112
ROWS_PER_TILE = N_PAD // NS


def _expand_body(x_ref, emb1_ref, emb2_ref, out_ref):
    xb = x_ref[:]
    for t in range(NUM_COMBO):
        e_row = emb1_ref[t // 3, :] + emb2_ref[t % 3, :]
        out_ref[t] = jnp.maximum(xb + e_row[None, :], 0.0)


def _expand(x, emb1, emb2):
    bn = 2000
    return pl.pallas_call(
        _expand_body,
        grid=(N_NODES // bn,),
        in_specs=[
            pl.BlockSpec((bn, EMB), lambda i: (i, 0)),
            pl.BlockSpec((5, EMB), lambda i: (0, 0)),
            pl.BlockSpec((3, EMB), lambda i: (0, 0)),
        ],
        out_specs=pl.BlockSpec((NUM_COMBO, bn, EMB), lambda i: (0, i, 0)),
        out_shape=jax.ShapeDtypeStruct((NUM_COMBO, N_NODES, EMB), jnp.float32),
    )(x, emb1, emb2)


def _sc_body(xp_hbm, gidx_hbm, dst_hbm, zeros_hbm, out_hbm,
             idx_c, dst_c, gbuf, agg_sh, gsem0, gsem1, gsem2, gsem3,
             ssem0, ssem1, ssem2, ssem3):
    c = lax.axis_index("c")
    s = lax.axis_index("s")
    wid = s * NC + c
    gsems = (gsem0, gsem1, gsem2, gsem3)
    ssems = (ssem0, ssem1, ssem2, ssem3)

    pltpu.sync_copy(zeros_hbm, agg_sh.at[pl.ds(s * ROWS_PER_TILE, ROWS_PER_TILE)])
    plsc.subcore_barrier()

    def chunk(ci, carry):
        pltpu.sync_copy(gidx_hbm.at[wid, ci], idx_c)
        pltpu.sync_copy(dst_hbm.at[wid, ci], dst_c)
        g = [None] * CHUNK
        sc = [None] * CHUNK
        for jj in range(CHUNK):
            slot = jj % NSLOT
            if jj >= NSLOT:
                sc[jj - NSLOT].wait()
            g[jj] = pltpu.async_copy(
                xp_hbm.at[idx_c.at[jj]], gbuf.at[slot], gsems[slot])
            if jj >= 1:
                pslot = (jj - 1) % NSLOT
                g[jj - 1].wait()
                sc[jj - 1] = pltpu.async_copy(
                    gbuf.at[pslot], agg_sh.at[dst_c.at[jj - 1]], ssems[pslot],
                    add=True)
        g[CHUNK - 1].wait()
        sc[CHUNK - 1] = pltpu.async_copy(
            gbuf.at[(CHUNK - 1) % NSLOT], agg_sh.at[dst_c.at[CHUNK - 1]],
            ssems[(CHUNK - 1) % NSLOT], add=True)
        for k in range(NSLOT):
            sc[CHUNK - 1 - k].wait()
        return carry

    lax.fori_loop(0, N_CHUNKS, chunk, 0)
    plsc.subcore_barrier()
    pltpu.sync_copy(agg_sh.at[pl.ds(s * ROWS_PER_TILE, ROWS_PER_TILE)],
                    out_hbm.at[c, pl.ds(s * ROWS_PER_TILE, ROWS_PER_TILE)])


def _scatter_agg(xplus, gidx, dst):
    zeros = jnp.zeros((ROWS_PER_TILE, EMB), jnp.float32)
    mesh = plsc.VectorSubcoreMesh(core_axis_name="c", subcore_axis_name="s",
                                  num_cores=NC)
    f = pl.kernel(
        _sc_body,
        out_type=jax.ShapeDtypeStruct((NC, N_PAD, EMB), jnp.float32),
        mesh=mesh,
        scratch_types=[
            pltpu.VMEM((CHUNK, EDGE_BATCH), jnp.int32),
            pltpu.VMEM((CHUNK, EDGE_BATCH), jnp.int32),
            pltpu.VMEM((NSLOT, EDGE_BATCH, EMB), jnp.float32),
            pltpu.VMEM_SHARED((N_PAD, EMB), jnp.float32),
            pltpu.SemaphoreType.DMA,
            pltpu.SemaphoreType.DMA,
            pltpu.SemaphoreType.DMA,
            pltpu.SemaphoreType.DMA,
            pltpu.SemaphoreType.DMA,
            pltpu.SemaphoreType.DMA,
            pltpu.SemaphoreType.DMA,
            pltpu.SemaphoreType.DMA,
        ],
    )
    return f(xplus, gidx, dst, zeros)


def _mlp_body(x_ref, parts_ref, w1_ref, b1_ref, w2_ref, b2_ref, eps_ref, out_ref):
    outv = x_ref[:] * (1.0 + eps_ref[0, 0]) + parts_ref[0] + parts_ref[1]
    h = lax.dot_general(outv, w1_ref[:], (((1,), (1,)), ((), ())),
                        preferred_element_type=jnp.float32)
    h = jnp.maximum(h + b1_ref[:], 0.0)
    y = lax.dot_general(h, w2_ref[:], (((1,), (1,)), ((), ())),
                        preferred_element_type=jnp.float32)
    out_ref[:] = y + b2_ref[:]


def _mlp(x, parts, W1, b1, W2, b2, eps):
    br = 1000
    return pl.pallas_call(
        _mlp_body,
        grid=(N_NODES // br,),
        in_specs=[
            pl.BlockSpec((br, EMB), lambda i: (i, 0)),
            pl.BlockSpec((NC, br, EMB), lambda i: (0, i, 0)),
            pl.BlockSpec((2 * EMB, EMB), lambda i: (0, 0)),
            pl.BlockSpec((1, 2 * EMB), lambda i: (0, 0)),
            pl.BlockSpec((EMB, 2 * EMB), lambda i: (0, 0)),
            pl.BlockSpec((1, EMB), lambda i: (0, 0)),
            pl.BlockSpec(memory_space=pltpu.SMEM),
        ],
        out_specs=pl.BlockSpec((br, EMB), lambda i: (i, 0)),
        out_shape=jax.ShapeDtypeStruct((N_NODES, EMB), jnp.float32),
    )(x, parts, W1, b1.reshape(1, -1), W2, b2.reshape(1, -1), eps.reshape(1, 1))


def kernel(x, edge_index, edge_attr, emb1, emb2, W1, b1, W2, b2, eps):
    ea = edge_attr.astype(jnp.int32)
    gidx32 = (ea[:, 0] * (3 * N_NODES) + ea[:, 1] * N_NODES
              + edge_index[0].astype(jnp.int32))
    gidx = gidx32.reshape(NW, N_CHUNKS, CHUNK, EDGE_BATCH)
    dst4d = edge_index[1].astype(jnp.int32).reshape(NW, N_CHUNKS, CHUNK, EDGE_BATCH)

    xplus = _expand(x, emb1, emb2).reshape(N_COMBO_ROWS, EMB)
    parts = _scatter_agg(xplus, gidx, dst4d)
    return _mlp(x, parts, W1, b1, W2, b2, eps)


# speedup vs baseline: 1.1753x; 1.1753x over previous
import jax
import jax.numpy as jnp
from jax import lax
from jax.experimental import pallas as pl
from jax.experimental.pallas import tpu as pltpu
from jax.experimental.pallas import tpu_sc as plsc

N_NODES = 10000
N_EDGES = 320000
EMB = 128
NUM_COMBO = 15
N_COMBO_ROWS = N_NODES * NUM_COMBO

NC = 2
NS = 16
NW = NC * NS
EDGE_BATCH = 80
N_BATCH = N_EDGES // EDGE_BATCH
NB_PER_TILE = N_BATCH // NW
CHUNK = 25
N_CHUNKS = NB_PER_TILE // CHUNK
NSLOT = 3
N_PAD = 10112
ROWS_PER_TILE = N_PAD // NS


def _expand_body(x_ref, emb1_ref, emb2_ref, out_ref):
    xb = x_ref[:]
    for t in range(NUM_COMBO):
        e_row = emb1_ref[t // 3, :] + emb2_ref[t % 3, :]
        out_ref[t] = jnp.maximum(xb + e_row[None, :], 0.0)


def _expand(x, emb1, emb2):
    bn = 2000
    return pl.pallas_call(
        _expand_body,
        grid=(N_NODES // bn,),
        in_specs=[
            pl.BlockSpec((bn, EMB), lambda i: (i, 0)),
            pl.BlockSpec((5, EMB), lambda i: (0, 0)),
            pl.BlockSpec((3, EMB), lambda i: (0, 0)),
        ],
        out_specs=pl.BlockSpec((NUM_COMBO, bn, EMB), lambda i: (0, i, 0)),
        out_shape=jax.ShapeDtypeStruct((NUM_COMBO, N_NODES, EMB), jnp.float32),
    )(x, emb1, emb2)


def _sc_body(xp_hbm, gidx_hbm, dst_hbm, zeros_hbm, out_hbm,
             idx_c, dst_c, gbuf, agg_sh, gsem0, gsem1, gsem2, gsem3,
             ssem0, ssem1, ssem2, ssem3):
    c = lax.axis_index("c")
    s = lax.axis_index("s")
    wid = s * NC + c
    gsems = (gsem0, gsem1, gsem2, gsem3)
    ssems = (ssem0, ssem1, ssem2, ssem3)

    pltpu.sync_copy(zeros_hbm, agg_sh.at[pl.ds(s * ROWS_PER_TILE, ROWS_PER_TILE)])
    plsc.subcore_barrier()

    def chunk(ci, carry):
        pltpu.sync_copy(gidx_hbm.at[wid, ci], idx_c)
        pltpu.sync_copy(dst_hbm.at[wid, ci], dst_c)
        g = [None] * CHUNK
        sc = [None] * CHUNK
        for jj in range(CHUNK):
            slot = jj % NSLOT
            if jj >= NSLOT:
                sc[jj - NSLOT].wait()
            g[jj] = pltpu.async_copy(
                xp_hbm.at[idx_c.at[jj]], gbuf.at[slot], gsems[slot])
            if jj >= 1:
                pslot = (jj - 1) % NSLOT
                g[jj - 1].wait()
                sc[jj - 1] = pltpu.async_copy(
                    gbuf.at[pslot], agg_sh.at[dst_c.at[jj - 1]], ssems[pslot],
                    add=True)
        g[CHUNK - 1].wait()
        sc[CHUNK - 1] = pltpu.async_copy(
            gbuf.at[(CHUNK - 1) % NSLOT], agg_sh.at[dst_c.at[CHUNK - 1]],
            ssems[(CHUNK - 1) % NSLOT], add=True)
        for k in range(NSLOT):
            sc[CHUNK - 1 - k].wait()
        return carry

    lax.fori_loop(0, N_CHUNKS, chunk, 0)
    plsc.subcore_barrier()
    pltpu.sync_copy(agg_sh.at[pl.ds(s * ROWS_PER_TILE, ROWS_PER_TILE)],
                    out_hbm.at[c, pl.ds(s * ROWS_PER_TILE, ROWS_PER_TILE)])


def _scatter_agg(xplus, gidx, dst):
    zeros = jnp.zeros((ROWS_PER_TILE, EMB), jnp.float32)
    mesh = plsc.VectorSubcoreMesh(core_axis_name="c", subcore_axis_name="s",
                                  num_cores=NC)
    f = pl.kernel(
        _sc_body,
        out_type=jax.ShapeDtypeStruct((NC, N_PAD, EMB), jnp.float32),
        mesh=mesh,
        scratch_types=[
            pltpu.VMEM((CHUNK, EDGE_BATCH), jnp.int32),
            pltpu.VMEM((CHUNK, EDGE_BATCH), jnp.int32),
            pltpu.VMEM((NSLOT, EDGE_BATCH, EMB), jnp.float32),
            pltpu.VMEM_SHARED((N_PAD, EMB), jnp.float32),
            pltpu.SemaphoreType.DMA,
            pltpu.SemaphoreType.DMA,
            pltpu.SemaphoreType.DMA,
            pltpu.SemaphoreType.DMA,
            pltpu.SemaphoreType.DMA,
            pltpu.SemaphoreType.DMA,
            pltpu.SemaphoreType.DMA,
            pltpu.SemaphoreType.DMA,
        ],
    )
    return f(xplus, gidx, dst, zeros)


def _mlp_body(x_ref, parts_ref, w1_ref, b1_ref, w2_ref, b2_ref, eps_ref, out_ref):
    outv = x_ref[:] * (1.0 + eps_ref[0, 0]) + parts_ref[0] + parts_ref[1]
    h = lax.dot_general(outv, w1_ref[:], (((1,), (1,)), ((), ())),
                        preferred_element_type=jnp.float32)
    h = jnp.maximum(h + b1_ref[:], 0.0)
    y = lax.dot_general(h, w2_ref[:], (((1,), (1,)), ((), ())),
                        preferred_element_type=jnp.float32)
    out_ref[:] = y + b2_ref[:]


def _mlp(x, parts, W1, b1, W2, b2, eps):
    br = 1000
    return pl.pallas_call(
        _mlp_body,
        grid=(N_NODES // br,),
        in_specs=[
            pl.BlockSpec((br, EMB), lambda i: (i, 0)),
            pl.BlockSpec((NC, br, EMB), lambda i: (0, i, 0)),
            pl.BlockSpec((2 * EMB, EMB), lambda i: (0, 0)),
            pl.BlockSpec((1, 2 * EMB), lambda i: (0, 0)),
            pl.BlockSpec((EMB, 2 * EMB), lambda i: (0, 0)),
            pl.BlockSpec((1, EMB), lambda i: (0, 0)),
            pl.BlockSpec(memory_space=pltpu.SMEM),
        ],
        out_specs=pl.BlockSpec((br, EMB), lambda i: (i, 0)),
        out_shape=jax.ShapeDtypeStruct((N_NODES, EMB), jnp.float32),
    )(x, parts, W1, b1.reshape(1, -1), W2, b2.reshape(1, -1), eps.reshape(1, 1))


def kernel(x, edge_index, edge_attr, emb1, emb2, W1, b1, W2, b2, eps):
    eat = edge_attr.astype(jnp.int32).T
    gidx32 = (eat[0] * (3 * N_NODES) + eat[1] * N_NODES
              + edge_index[0].astype(jnp.int32))
    gidx = gidx32.reshape(NW, N_CHUNKS, CHUNK, EDGE_BATCH)
    dst4d = edge_index[1].astype(jnp.int32).reshape(NW, N_CHUNKS, CHUNK, EDGE_BATCH)

    xplus = _expand(x, emb1, emb2).reshape(N_COMBO_ROWS, EMB)
    parts = _scatter_agg(xplus, gidx, dst4d)
    return _mlp(x, parts, W1, b1, W2, b2, eps)


# speedup vs baseline: 1.2331x; 1.0492x over previous
import jax
import jax.numpy as jnp
from jax import lax
from jax.experimental import pallas as pl
from jax.experimental.pallas import tpu as pltpu
from jax.experimental.pallas import tpu_sc as plsc

N_NODES = 10000
N_EDGES = 320000
EMB = 128
NUM_COMBO = 15
N_COMBO_ROWS = N_NODES * NUM_COMBO

NC = 2
NS = 16
NW = NC * NS
EDGE_BATCH = 80
N_BATCH = N_EDGES // EDGE_BATCH
NB_PER_TILE = N_BATCH // NW
CHUNK = 25
N_CHUNKS = NB_PER_TILE // CHUNK
NSLOT = 3
N_PAD = 10112
ROWS_PER_TILE = N_PAD // NS


def _expand_body(x_ref, emb1_ref, emb2_ref, out_ref):
    xb = x_ref[:]
    for t in range(NUM_COMBO):
        e_row = emb1_ref[t // 3, :] + emb2_ref[t % 3, :]
        out_ref[t] = jnp.maximum(xb + e_row[None, :], 0.0)


def _expand(x, emb1, emb2):
    bn = 2000
    return pl.pallas_call(
        _expand_body,
        grid=(N_NODES // bn,),
        in_specs=[
            pl.BlockSpec((bn, EMB), lambda i: (i, 0)),
            pl.BlockSpec((5, EMB), lambda i: (0, 0)),
            pl.BlockSpec((3, EMB), lambda i: (0, 0)),
        ],
        out_specs=pl.BlockSpec((NUM_COMBO, bn, EMB), lambda i: (0, i, 0)),
        out_shape=jax.ShapeDtypeStruct((NUM_COMBO, N_NODES, EMB), jnp.float32),
    )(x, emb1, emb2)


def _sc_body(xp_hbm, gidx_hbm, dst_hbm, zeros_hbm, out_hbm,
             idx0, idx1, dst0, dst1, gbuf, agg_sh, gsem0, gsem1, gsem2,
             ssem0, ssem1, ssem2, dsem, isem):
    c = lax.axis_index("c")
    s = lax.axis_index("s")
    wid = s * NC + c
    gsems = (gsem0, gsem1, gsem2)
    ssems = (ssem0, ssem1, ssem2)
    dbufs = (dst0, dst1)
    ibufs = (idx0, idx1)

    pltpu.sync_copy(zeros_hbm, agg_sh.at[pl.ds(s * ROWS_PER_TILE, ROWS_PER_TILE)])
    pltpu.sync_copy(gidx_hbm.at[wid, 0], idx0)
    pltpu.sync_copy(dst_hbm.at[wid, 0], dst0)
    plsc.subcore_barrier()

    g = [None] * NB_PER_TILE
    sc = [None] * NB_PER_TILE
    dpref = [None] * N_CHUNKS
    ipref = [None] * N_CHUNKS

    def fire_scatter(j):
        ci = j // CHUNK
        if j % CHUNK == 0 and ci > 0:
            dpref[ci].wait()
        sc[j] = pltpu.async_copy(
            gbuf.at[j % NSLOT],
            agg_sh.at[dbufs[ci % 2].at[j % CHUNK]],
            ssems[j % NSLOT], add=True)

    for j in range(NB_PER_TILE):
        ci = j // CHUNK
        if j >= NSLOT:
            sc[j - NSLOT].wait()
        if j % CHUNK == 0 and ci > 0:
            ipref[ci].wait()
        g[j] = pltpu.async_copy(
            xp_hbm.at[ibufs[ci % 2].at[j % CHUNK]], gbuf.at[j % NSLOT],
            gsems[j % NSLOT])
        if j % CHUNK == NSLOT and ci + 1 < N_CHUNKS:
            ipref[ci + 1] = pltpu.async_copy(
                gidx_hbm.at[wid, ci + 1], ibufs[(ci + 1) % 2], isem)
            dpref[ci + 1] = pltpu.async_copy(
                dst_hbm.at[wid, ci + 1], dbufs[(ci + 1) % 2], dsem)
        if j >= 1:
            g[j - 1].wait()
            fire_scatter(j - 1)
    g[NB_PER_TILE - 1].wait()
    fire_scatter(NB_PER_TILE - 1)
    for k in range(NSLOT):
        sc[NB_PER_TILE - 1 - k].wait()

    plsc.subcore_barrier()
    pltpu.sync_copy(agg_sh.at[pl.ds(s * ROWS_PER_TILE, ROWS_PER_TILE)],
                    out_hbm.at[c, pl.ds(s * ROWS_PER_TILE, ROWS_PER_TILE)])


def _scatter_agg(xplus, gidx, dst):
    zeros = jnp.zeros((ROWS_PER_TILE, EMB), jnp.float32)
    mesh = plsc.VectorSubcoreMesh(core_axis_name="c", subcore_axis_name="s",
                                  num_cores=NC)
    f = pl.kernel(
        _sc_body,
        out_type=jax.ShapeDtypeStruct((NC, N_PAD, EMB), jnp.float32),
        mesh=mesh,
        scratch_types=[
            pltpu.VMEM((CHUNK, EDGE_BATCH), jnp.int32),
            pltpu.VMEM((CHUNK, EDGE_BATCH), jnp.int32),
            pltpu.VMEM((CHUNK, EDGE_BATCH), jnp.int32),
            pltpu.VMEM((CHUNK, EDGE_BATCH), jnp.int32),
            pltpu.VMEM((NSLOT, EDGE_BATCH, EMB), jnp.float32),
            pltpu.VMEM_SHARED((N_PAD, EMB), jnp.float32),
            pltpu.SemaphoreType.DMA,
            pltpu.SemaphoreType.DMA,
            pltpu.SemaphoreType.DMA,
            pltpu.SemaphoreType.DMA,
            pltpu.SemaphoreType.DMA,
            pltpu.SemaphoreType.DMA,
            pltpu.SemaphoreType.DMA,
            pltpu.SemaphoreType.DMA,
        ],
    )
    return f(xplus, gidx, dst, zeros)


def _mlp_body(x_ref, parts_ref, w1_ref, b1_ref, w2_ref, b2_ref, eps_ref, out_ref):
    outv = x_ref[:] * (1.0 + eps_ref[0, 0]) + parts_ref[0] + parts_ref[1]
    h = lax.dot_general(outv, w1_ref[:], (((1,), (1,)), ((), ())),
                        preferred_element_type=jnp.float32)
    h = jnp.maximum(h + b1_ref[:], 0.0)
    y = lax.dot_general(h, w2_ref[:], (((1,), (1,)), ((), ())),
                        preferred_element_type=jnp.float32)
    out_ref[:] = y + b2_ref[:]


def _mlp(x, parts, W1, b1, W2, b2, eps):
    br = 1000
    return pl.pallas_call(
        _mlp_body,
        grid=(N_NODES // br,),
        in_specs=[
            pl.BlockSpec((br, EMB), lambda i: (i, 0)),
            pl.BlockSpec((NC, br, EMB), lambda i: (0, i, 0)),
            pl.BlockSpec((2 * EMB, EMB), lambda i: (0, 0)),
            pl.BlockSpec((1, 2 * EMB), lambda i: (0, 0)),
            pl.BlockSpec((EMB, 2 * EMB), lambda i: (0, 0)),
            pl.BlockSpec((1, EMB), lambda i: (0, 0)),
            pl.BlockSpec(memory_space=pltpu.SMEM),
        ],
        out_specs=pl.BlockSpec((br, EMB), lambda i: (i, 0)),
        out_shape=jax.ShapeDtypeStruct((N_NODES, EMB), jnp.float32),
    )(x, parts, W1, b1.reshape(1, -1), W2, b2.reshape(1, -1), eps.reshape(1, 1))


def kernel(x, edge_index, edge_attr, emb1, emb2, W1, b1, W2, b2, eps):
    eat = edge_attr.astype(jnp.int32).T
    gidx32 = (eat[0] * (3 * N_NODES) + eat[1] * N_NODES
              + edge_index[0].astype(jnp.int32))
    gidx = gidx32.reshape(NW, N_CHUNKS, CHUNK, EDGE_BATCH)
    dst4d = edge_index[1].astype(jnp.int32).reshape(NW, N_CHUNKS, CHUNK, EDGE_BATCH)

    xplus = _expand(x, emb1, emb2).reshape(N_COMBO_ROWS, EMB)
    parts = _scatter_agg(xplus, gidx, dst4d)
    return _mlp(x, parts, W1, b1, W2, b2, eps)
